# baseline (speedup 1.0000x reference)
import functools

import jax
import jax.numpy as jnp
from jax import lax
from jax.experimental import pallas as pl
from jax.experimental.pallas import tpu as pltpu
from jax.experimental.pallas import tpu_sc as plsc

VOCAB_SIZE = 100000
EMB_D = 64
BATCH = 1024

NUM_WORKERS = 32
BV = 4096
TPB = 2944


HALF_V = 50048
N_TPB = HALF_V // TPB


def _tc_transpose_pairs(table_t):
    def tr_kernel(lo_ref, hi_ref, o_ref):
        o_ref[:, :EMB_D] = jnp.swapaxes(lo_ref[...], 0, 1)
        o_ref[:, EMB_D:] = jnp.swapaxes(hi_ref[...], 0, 1)

    return pl.pallas_call(
        tr_kernel,
        grid=(N_TPB,),
        in_specs=[
            pl.BlockSpec((EMB_D, TPB), lambda j: (0, j)),
            pl.BlockSpec((EMB_D, TPB), lambda j: (0, j + N_TPB)),
        ],
        out_specs=pl.BlockSpec((TPB, 2 * EMB_D), lambda j: (j, 0)),
        out_shape=jax.ShapeDtypeStruct((HALF_V, 2 * EMB_D), jnp.float32),
        compiler_params=pltpu.CompilerParams(
            dimension_semantics=("arbitrary",),
        ),
    )(table_t, table_t)


def _sc_gather_pairs(table_pairs, pair_idx):
    b_per_w = BATCH // NUM_WORKERS
    mesh = plsc.VectorSubcoreMesh(core_axis_name="c", subcore_axis_name="s")

    @functools.partial(
        pl.kernel,
        mesh=mesh,
        out_type=jax.ShapeDtypeStruct((BATCH, 2 * EMB_D), jnp.float32),
        scratch_types=[
            pltpu.VMEM((b_per_w,), jnp.int32),
            pltpu.VMEM((b_per_w, 2 * EMB_D), jnp.float32),
            pltpu.SemaphoreType.DMA,
        ],
    )
    def gather_kernel(table_hbm, idx_hbm, out_hbm, idx_v, rows_v, sem):
        wid = lax.axis_index("s") * 2 + lax.axis_index("c")
        base = wid * b_per_w
        pltpu.sync_copy(idx_hbm.at[pl.ds(base, b_per_w)], idx_v)
        pltpu.async_copy(table_hbm.at[idx_v], rows_v, sem).wait()
        pltpu.sync_copy(rows_v, out_hbm.at[pl.ds(base, b_per_w)])

    return gather_kernel(table_pairs, pair_idx)


def _tc_matmul_t(pair_embeds, parity, WT, b1):
    nv = pl.cdiv(VOCAB_SIZE, BV)

    def mm_kernel(e_ref, p_ref, wt_ref, b_ref, o_ref, e64):
        @pl.when(pl.program_id(0) == 0)
        def _():
            pair = e_ref[...]
            p = p_ref[...]
            e64[...] = jnp.where(p > 0.5, pair[:, EMB_D:], pair[:, :EMB_D])

        acc = lax.dot_general(
            wt_ref[...], e64[...],
            (((0,), (1,)), ((), ())),
            preferred_element_type=jnp.float32,
        )
        o_ref[...] = acc + jnp.swapaxes(b_ref[...], 0, 1)

    return pl.pallas_call(
        mm_kernel,
        grid=(nv,),
        in_specs=[
            pl.BlockSpec((BATCH, 2 * EMB_D), lambda j: (0, 0)),
            pl.BlockSpec((BATCH, 1), lambda j: (0, 0)),
            pl.BlockSpec((EMB_D, BV), lambda j: (0, j)),
            pl.BlockSpec((1, BV), lambda j: (0, j)),
        ],
        out_specs=pl.BlockSpec((BV, BATCH), lambda j: (j, 0)),
        out_shape=jax.ShapeDtypeStruct((VOCAB_SIZE, BATCH), jnp.float32),
        scratch_shapes=[pltpu.VMEM((BATCH, EMB_D), jnp.float32)],
        compiler_params=pltpu.CompilerParams(
            dimension_semantics=("arbitrary",),
            fuse_transposed_lhs_in_matmul=True,
        ),
    )(pair_embeds, parity, WT, b1)


def kernel(inputs, table, W, b):
    idx = inputs.astype(jnp.int32)
    hi = (idx >= HALF_V).astype(jnp.int32)
    pair_idx = idx - hi * HALF_V
    parity = hi.astype(jnp.float32).reshape(BATCH, 1)
    table_pairs = _tc_transpose_pairs(jnp.transpose(table))
    pair_embeds = _sc_gather_pairs(table_pairs, pair_idx)
    logits_t = _tc_matmul_t(
        pair_embeds, parity, jnp.transpose(W), b.reshape(1, VOCAB_SIZE))
    return jnp.transpose(logits_t)
